# speedup vs baseline: 4.7870x; 4.7870x over previous
import functools

import jax
import jax.numpy as jnp
from jax import lax
from jax.experimental import pallas as pl
from jax.experimental.pallas import tpu as pltpu
from jax.experimental.pallas import tpu_sc as plsc

_N = 10000
_NP = 10240
_E = 320000
_D = 128

_NC = 2
_NS = 16
_NW = _NC * _NS
_EPW = _E // _NW
_K = 80
_CHUNKS = _EPW // _K
_RPT = _NP // _NS


def _segsum_kernel(table, src, dst, zeros, out, acc, srcv, dstv, rows, sem):
    c = lax.axis_index("c")
    s = lax.axis_index("s")
    pltpu.sync_copy(zeros, acc.at[pl.ds(s * _RPT, _RPT)])
    plsc.subcore_barrier()

    base = (c * _NS + s) * _EPW

    def body(j, carry):
        off = base + j * _K
        pltpu.sync_copy(src.at[pl.ds(off, _K)], srcv)
        pltpu.sync_copy(dst.at[pl.ds(off, _K)], dstv)
        pltpu.async_copy(table.at[srcv], rows, sem).wait()
        pltpu.sync_copy(rows, acc.at[dstv], add=True)
        return carry

    lax.fori_loop(0, _CHUNKS, body, 0)
    plsc.subcore_barrier()
    pltpu.sync_copy(acc.at[pl.ds(s * _RPT, _RPT)],
                    out.at[c, pl.ds(s * _RPT, _RPT)])


_segsum = functools.partial(
    pl.kernel,
    out_type=jax.ShapeDtypeStruct((_NC, _NP, _D), jnp.float32),
    mesh=plsc.VectorSubcoreMesh(core_axis_name="c", subcore_axis_name="s"),
    scratch_types=[
        pltpu.VMEM_SHARED((_NP, _D), jnp.float32),
        pltpu.VMEM((_K,), jnp.int32),
        pltpu.VMEM((_K,), jnp.int32),
        pltpu.VMEM((_K, _D), jnp.float32),
        pltpu.SemaphoreType.DMA,
    ],
)(_segsum_kernel)


_R = 256


def _dense_body(p_ref, x_ref, wr_ref, wt_ref, b_ref, o_ref):
    agg = p_ref[0] + p_ref[1]
    acc = lax.dot_general(agg, wr_ref[...], (((1,), (1,)), ((), ())),
                          preferred_element_type=jnp.float32)
    acc = acc + lax.dot_general(x_ref[...], wt_ref[...],
                                (((1,), (1,)), ((), ())),
                                preferred_element_type=jnp.float32)
    o_ref[...] = jnp.maximum(acc + b_ref[...], 0.0)


def _dense_heads_body(p_ref, x_ref, wr_ref, wt_ref, b_ref, wh_ref, bh_ref,
                      o_ref):
    agg = p_ref[0] + p_ref[1]
    acc = lax.dot_general(agg, wr_ref[...], (((1,), (1,)), ((), ())),
                          preferred_element_type=jnp.float32)
    acc = acc + lax.dot_general(x_ref[...], wt_ref[...],
                                (((1,), (1,)), ((), ())),
                                preferred_element_type=jnp.float32)
    h = jnp.maximum(acc + b_ref[...], 0.0)
    o_ref[...] = lax.dot_general(h, wh_ref[...], (((1,), (1,)), ((), ())),
                                 preferred_element_type=jnp.float32) + bh_ref[...]


def _dense_layer(parts, xp, w_rel, w_root, b):
    return pl.pallas_call(
        _dense_body,
        grid=(_NP // _R,),
        in_specs=[
            pl.BlockSpec((_NC, _R, _D), lambda i: (0, i, 0)),
            pl.BlockSpec((_R, _D), lambda i: (i, 0)),
            pl.BlockSpec((_D, _D), lambda i: (0, 0)),
            pl.BlockSpec((_D, _D), lambda i: (0, 0)),
            pl.BlockSpec((1, _D), lambda i: (0, 0)),
        ],
        out_specs=pl.BlockSpec((_R, _D), lambda i: (i, 0)),
        out_shape=jax.ShapeDtypeStruct((_NP, _D), jnp.float32),
    )(parts, xp, w_rel, w_root, b)


def _dense_layer_heads(parts, xp, w_rel, w_root, b, w_heads, b_heads):
    return pl.pallas_call(
        _dense_heads_body,
        grid=(_NP // _R,),
        in_specs=[
            pl.BlockSpec((_NC, _R, _D), lambda i: (0, i, 0)),
            pl.BlockSpec((_R, _D), lambda i: (i, 0)),
            pl.BlockSpec((_D, _D), lambda i: (0, 0)),
            pl.BlockSpec((_D, _D), lambda i: (0, 0)),
            pl.BlockSpec((1, _D), lambda i: (0, 0)),
            pl.BlockSpec((_D, _D), lambda i: (0, 0)),
            pl.BlockSpec((1, _D), lambda i: (0, 0)),
        ],
        out_specs=pl.BlockSpec((_R, _D), lambda i: (i, 0)),
        out_shape=jax.ShapeDtypeStruct((_NP, _D), jnp.float32),
    )(parts, xp, w_rel, w_root, b, w_heads, b_heads)


def kernel(x, edge_index, W1_rel, b1, W1_root, W2_rel, b2, W2_root,
           Wa, ba, Wo, bo):
    src = edge_index[0]
    dst = edge_index[1]
    zeros = jnp.zeros((_RPT, _D), jnp.float32)
    xp = jnp.pad(x, ((0, _NP - _N), (0, 0)))

    w_heads = jnp.zeros((_D, _D), jnp.float32)
    w_heads = w_heads.at[:2, :].set(Wa).at[2, :].set(Wo[0])
    b_heads = jnp.zeros((_D,), jnp.float32)
    b_heads = b_heads.at[:2].set(ba).at[2].set(bo[0])

    parts1 = _segsum(x, src, dst, zeros)
    h1 = _dense_layer(parts1, xp, W1_rel, W1_root, b1.reshape(1, _D))
    parts2 = _segsum(h1, src, dst, zeros)
    out = _dense_layer_heads(parts2, h1, W2_rel, W2_root, b2.reshape(1, _D),
                             w_heads, b_heads.reshape(1, _D))
    task_allocation = out[:_N, :2]
    task_order = out[:_N, 2:3]
    return (task_allocation, task_order)
